# Initial kernel scaffold; baseline (speedup 1.0000x reference)
import jax
import jax.numpy as jnp
from jax.experimental import pallas as pl


def kernel(h_i, R_i, t_i, v_i, entity_emb, relation_emb):
    raise NotImplementedError("write your pallas kernel here")


# speedup vs baseline: 1.0459x; 1.0459x over previous
import jax
import jax.numpy as jnp
from jax import lax
from jax.experimental import pallas as pl
from jax.experimental.pallas import tpu as pltpu
from jax.experimental.pallas import tpu_sc as plsc

DIM = 16
RELDIM = 256

NC = 2
NS = 16
NW = NC * NS

E_TOTAL = 2 * 1024 * 2 * 32 + 1024
E_PER_W = E_TOTAL // NW
R_TOTAL = 1024 * 2 * 32
R_PER_W = R_TOTAL // NW
R_CHUNK = 128
R_NCHUNK = R_PER_W // R_CHUNK


def _gather_body(ent_hbm, eidx_hbm, rel_hbm, ridx_hbm,
                 eout_hbm, rout_hbm,
                 eidx_v, erows_v, ridx_v, rrows_v, sem):
    wid = lax.axis_index("s") * NC + lax.axis_index("c")

    ebase = wid * E_PER_W
    pltpu.sync_copy(eidx_hbm.at[pl.ds(ebase, E_PER_W)], eidx_v)
    pltpu.async_copy(ent_hbm.at[eidx_v], erows_v, sem).wait()
    pltpu.sync_copy(erows_v, eout_hbm.at[pl.ds(ebase, E_PER_W)])

    pltpu.sync_copy(ridx_hbm.at[wid], ridx_v)
    rbase = wid * R_PER_W
    for j in range(R_NCHUNK):
        pltpu.async_copy(rel_hbm.at[ridx_v.at[j]], rrows_v, sem).wait()
        pltpu.sync_copy(rrows_v, rout_hbm.at[pl.ds(rbase + j * R_CHUNK, R_CHUNK)])


_gather_call_cache = []


def _gather_call():
    if not _gather_call_cache:
        mesh = plsc.VectorSubcoreMesh(core_axis_name="c", subcore_axis_name="s",
                                      num_cores=NC, num_subcores=NS)
        _gather_call_cache.append(pl.kernel(
            _gather_body,
            out_type=(
                jax.ShapeDtypeStruct((E_TOTAL, DIM), jnp.float32),
                jax.ShapeDtypeStruct((R_TOTAL, RELDIM), jnp.float32),
            ),
            mesh=mesh,
            scratch_types=[
                pltpu.VMEM((E_PER_W,), jnp.int32),
                pltpu.VMEM((E_PER_W, DIM), jnp.float32),
                pltpu.VMEM((R_NCHUNK, R_CHUNK), jnp.int32),
                pltpu.VMEM((R_CHUNK, RELDIM), jnp.float32),
                pltpu.SemaphoreType.DMA,
            ],
            compiler_params=pltpu.CompilerParams(use_tc_tiling_on_sc=False),
        ))
    return _gather_call_cache[0]


def kernel(h_i, R_i, t_i, v_i, entity_emb, relation_emb):
    batch_size, n_hop, n_memory = h_i.shape
    n_ht = batch_size * n_hop * n_memory

    eidx = jnp.concatenate([h_i.reshape(-1), t_i.reshape(-1), v_i])
    ridx = R_i.reshape(NW, R_NCHUNK, R_CHUNK)
    rel2d = relation_emb.reshape(-1, RELDIM)

    erows, rrows = _gather_call()(entity_emb, eidx, rel2d, ridx)

    hs = erows[:n_ht].reshape(batch_size, n_hop, n_memory, DIM)
    ts = erows[n_ht:2 * n_ht].reshape(batch_size, n_hop, n_memory, DIM)
    vs = erows[2 * n_ht:]
    Rs = rrows.reshape(batch_size, n_hop, n_memory, DIM, DIM)
    return (hs, Rs, ts, vs)
